# baseline (speedup 1.0000x reference)
import jax
import jax.numpy as jnp
from jax.experimental import pallas as pl

_N, _D, _E, _K = 8192, 2048, 16, 2
_EP = 128
_TN = 512


def _moe_body(xb_ref, wr_ref, br_ref, we_ref, be_ref, out_ref, loss_ref):
    n = pl.program_id(0)
    xb = xb_ref[...]

    logits = jax.lax.dot_general(
        xb, wr_ref[...], (((1,), (1,)), ((), ())),
        preferred_element_type=jnp.float32)
    logits = logits + br_ref[...]
    m = jnp.max(logits, axis=-1, keepdims=True)
    e = jnp.exp(logits - m)
    s = jnp.sum(e, axis=-1, keepdims=True)
    m1 = jnp.max(e, axis=-1, keepdims=True)
    lane = jax.lax.broadcasted_iota(jnp.int32, (_TN, _EP), 1)
    first_idx = jnp.min(jnp.where(e == m1, lane, _EP), axis=-1, keepdims=True)
    e_masked = jnp.where(lane == first_idx, -jnp.inf, e)
    m2 = jnp.max(e_masked, axis=-1, keepdims=True)
    tot = m1 + m2
    w0 = m1 / tot
    w1 = m2 / tot

    part = jnp.sum(tot / s) * (1.0 / _N)

    @pl.when(n == 0)
    def _init():
        loss_ref[...] = jnp.zeros_like(loss_ref)

    loss_ref[...] = loss_ref[...] + part

    a0 = jax.lax.dot_general(
        xb, we_ref[0], (((1,), (1,)), ((), ())),
        preferred_element_type=jnp.float32)
    a1 = jax.lax.dot_general(
        xb, we_ref[1], (((1,), (1,)), ((), ())),
        preferred_element_type=jnp.float32)
    out_ref[...] = (w0 * a0 + w1 * a1
                    + w0 * be_ref[0:1, :] + w1 * be_ref[1:2, :])


def kernel(x, Wr, br, We, be):
    xb = x.astype(jnp.bfloat16)
    wr_p = jnp.zeros((_EP, _D), jnp.bfloat16).at[:_E].set(Wr.astype(jnp.bfloat16))
    br_p = jnp.full((1, _EP), -1e30, jnp.float32).at[0, :_E].set(br)
    we_b = We.astype(jnp.bfloat16)

    out, loss = pl.pallas_call(
        _moe_body,
        grid=(_N // _TN,),
        in_specs=[
            pl.BlockSpec((_TN, _D), lambda n: (n, 0)),
            pl.BlockSpec((_EP, _D), lambda n: (0, 0)),
            pl.BlockSpec((1, _EP), lambda n: (0, 0)),
            pl.BlockSpec((_K, _D, _D), lambda n: (0, 0, 0)),
            pl.BlockSpec((_K, _D), lambda n: (0, 0)),
        ],
        out_specs=[
            pl.BlockSpec((_TN, _D), lambda n: (n, 0)),
            pl.BlockSpec((1, 1), lambda n: (0, 0)),
        ],
        out_shape=[
            jax.ShapeDtypeStruct((_N, _D), jnp.float32),
            jax.ShapeDtypeStruct((1, 1), jnp.float32),
        ],
    )(xb, wr_p, br_p, we_b, be)
    return out, loss[0, 0]
